# baseline (speedup 1.0000x reference)
import functools

import jax
import jax.numpy as jnp
from jax import lax
from jax.experimental import pallas as pl
from jax.experimental.pallas import tpu as pltpu
from jax.experimental.pallas import tpu_sc as plsc

N_ = 10000
F_ = 320000
C_ = 128
NC_ = 2
NS_ = 16
NW_ = NC_ * NS_
CHUNK_ = 128
FH_ = F_ // 4
NFULL_ = 19
NEXTRA_ = (FH_ - NW_ * NFULL_ * CHUNK_) // CHUNK_
ROWS_W_ = NFULL_ * CHUNK_

BT_ = 2000
NB_ = F_ // BT_
NBH_ = FH_ // BT_


def _c_body(inp, sw, t):
  x = inp[...]
  t[pl.ds(0, N_), :] = x * sw[0, :][None, :]
  t[pl.ds(N_, N_), :] = x * sw[1, :][None, :]
  t[pl.ds(2 * N_, N_), :] = x * sw[2, :][None, :]


def _sc_body(base0, t_hbm, i0_hbm, i1_hbm, i2_hbm, v2f_hbm,
             iv0, iv1, iv2,
             ra0, ra1, ra2, rb0, rb1, rb2, sa, sb):
  wid = lax.axis_index("s") * NC_ + lax.axis_index("c")
  wbase = wid * ROWS_W_
  gbase = base0 + wbase

  pltpu.sync_copy(i0_hbm.at[pl.ds(gbase, ROWS_W_)], iv0)
  pltpu.sync_copy(i1_hbm.at[pl.ds(gbase, ROWS_W_)], iv1)
  pltpu.sync_copy(i2_hbm.at[pl.ds(gbase, ROWS_W_)], iv2)

  def issue(bufs, sem, j):
    off = j * CHUNK_
    pltpu.async_copy(t_hbm.at[iv0.at[pl.ds(off, CHUNK_)]], bufs[0], sem)
    pltpu.async_copy(t_hbm.at[iv1.at[pl.ds(off, CHUNK_)]], bufs[1], sem)
    pltpu.async_copy(t_hbm.at[iv2.at[pl.ds(off, CHUNK_)]], bufs[2], sem)

  def drain(bufs, sem, j):
    off = j * CHUNK_
    pltpu.make_async_copy(t_hbm.at[iv0.at[pl.ds(off, CHUNK_)]], bufs[0],
                          sem).wait()
    pltpu.make_async_copy(t_hbm.at[iv1.at[pl.ds(off, CHUNK_)]], bufs[1],
                          sem).wait()
    pltpu.make_async_copy(t_hbm.at[iv2.at[pl.ds(off, CHUNK_)]], bufs[2],
                          sem).wait()

  def combine(bufs):
    def row(r, carry):
      for s in range(C_ // 16):
        sl = pl.ds(s * 16, 16)
        bufs[0][r, sl] = bufs[0][r, sl] + bufs[1][r, sl] + bufs[2][r, sl]
      return carry

    lax.fori_loop(0, CHUNK_, row, 0)

  def store(bufs, base):
    pltpu.sync_copy(bufs[0], v2f_hbm.at[pl.ds(base, CHUNK_)])

  bufs_a = (ra0, ra1, ra2)
  bufs_b = (rb0, rb1, rb2)

  issue(bufs_a, sa, 0)

  def body(i, carry):
    j0 = 2 * i

    @pl.when(j0 + 1 < NFULL_)
    def _():
      issue(bufs_b, sb, j0 + 1)

    drain(bufs_a, sa, j0)
    combine(bufs_a)
    store(bufs_a, wbase + j0 * CHUNK_)

    @pl.when(j0 + 2 < NFULL_)
    def _():
      issue(bufs_a, sa, j0 + 2)

    @pl.when(j0 + 1 < NFULL_)
    def _():
      drain(bufs_b, sb, j0 + 1)
      combine(bufs_b)
      store(bufs_b, wbase + (j0 + 1) * CHUNK_)
    return carry

  lax.fori_loop(0, (NFULL_ + 1) // 2, body, 0)

  @pl.when(wid < NEXTRA_)
  def _():
    base = (NW_ * NFULL_ + wid) * CHUNK_
    pltpu.sync_copy(i0_hbm.at[pl.ds(base0 + base, CHUNK_)],
                    iv0.at[pl.ds(0, CHUNK_)])
    pltpu.sync_copy(i1_hbm.at[pl.ds(base0 + base, CHUNK_)],
                    iv1.at[pl.ds(0, CHUNK_)])
    pltpu.sync_copy(i2_hbm.at[pl.ds(base0 + base, CHUNK_)],
                    iv2.at[pl.ds(0, CHUNK_)])
    issue(bufs_a, sa, 0)
    drain(bufs_a, sa, 0)
    combine(bufs_a)
    store(bufs_a, base)


@functools.lru_cache(maxsize=None)
def _get_sc_combine(base0):
  return pl.kernel(
    out_type=jax.ShapeDtypeStruct((FH_, C_), jnp.float32),
    mesh=plsc.VectorSubcoreMesh(core_axis_name="c", subcore_axis_name="s"),
    scratch_types=[
        pltpu.VMEM((ROWS_W_,), jnp.int32),
        pltpu.VMEM((ROWS_W_,), jnp.int32),
        pltpu.VMEM((ROWS_W_,), jnp.int32),
        pltpu.VMEM((CHUNK_, C_), jnp.float32),
        pltpu.VMEM((CHUNK_, C_), jnp.float32),
        pltpu.VMEM((CHUNK_, C_), jnp.float32),
        pltpu.VMEM((CHUNK_, C_), jnp.float32),
        pltpu.VMEM((CHUNK_, C_), jnp.float32),
        pltpu.VMEM((CHUNK_, C_), jnp.float32),
        pltpu.SemaphoreType.DMA,
        pltpu.SemaphoreType.DMA,
    ],
  )(functools.partial(_sc_body, base0))


def _a_body(v2f, dw, bb, stats):
  acc = jnp.dot(v2f[...], dw[...], preferred_element_type=jnp.float32)
  acc = acc + bb[0, :][None, :]
  r = jnp.maximum(acc, 0.0)
  s = jnp.sum(r, axis=0)
  s2 = jnp.sum(r * r, axis=0)
  upd = jnp.concatenate(
      [s[None, :], s2[None, :], jnp.zeros((6, C_), jnp.float32)], axis=0)

  @pl.when(pl.program_id(0) == 0)
  def _():
    stats[...] = upd

  @pl.when(pl.program_id(0) != 0)
  def _():
    stats[...] = stats[...] + upd


def _b_body(vh1, vh2, vh3, vh4, dw, bb, st1, st2, st3, st4, gb, out):
  i = pl.program_id(0)
  x = jnp.where(i < NBH_, vh1[...],
                jnp.where(i < 2 * NBH_, vh2[...],
                          jnp.where(i < 3 * NBH_, vh3[...], vh4[...])))
  acc = jnp.dot(x, dw[...], preferred_element_type=jnp.float32)
  acc = acc + bb[0, :][None, :]
  r = jnp.maximum(acc, 0.0)
  s = st1[0, :] + st2[0, :] + st3[0, :] + st4[0, :]
  s2 = st1[1, :] + st2[1, :] + st3[1, :] + st4[1, :]
  mean = s / F_
  var = s2 / F_ - mean * mean
  inv = gb[0, :] / jnp.sqrt(var + 1e-5)
  out[...] = (r - mean[None, :]) * inv[None, :] + gb[1, :][None, :]


def kernel(inputs, face, spatial_weights, depth_weights, biases,
           bn_gamma, bn_beta):
  face32 = face.astype(jnp.int32)
  ft = face32.T
  i0 = ft[0]
  i1 = ft[1] + N_
  i2 = ft[2] + 2 * N_

  sw8 = jnp.pad(spatial_weights[:, :, 0], ((0, 5), (0, 0)))
  bb8 = jnp.pad(biases, ((0, 7), (0, 0)))
  gb8 = jnp.pad(jnp.stack([bn_gamma, bn_beta]), ((0, 6), (0, 0)))

  t = pl.pallas_call(
      _c_body,
      in_specs=[
          pl.BlockSpec((N_, C_), lambda: (0, 0)),
          pl.BlockSpec((8, C_), lambda: (0, 0)),
      ],
      out_specs=pl.BlockSpec((3 * N_, C_), lambda: (0, 0)),
      out_shape=jax.ShapeDtypeStruct((3 * N_, C_), jnp.float32),
  )(inputs, sw8)

  vhs = [_get_sc_combine(q * FH_)(t, i0, i1, i2) for q in range(4)]

  def run_a(vh):
    return pl.pallas_call(
        _a_body,
        grid=(NBH_,),
        in_specs=[
            pl.BlockSpec((BT_, C_), lambda i: (i, 0)),
            pl.BlockSpec((C_, C_), lambda i: (0, 0)),
            pl.BlockSpec((8, C_), lambda i: (0, 0)),
        ],
        out_specs=pl.BlockSpec((8, C_), lambda i: (0, 0)),
        out_shape=jax.ShapeDtypeStruct((8, C_), jnp.float32),
    )(vh, depth_weights, bb8)

  sts = [run_a(vh) for vh in vhs]

  def qmap(q):
    return lambda i: (jnp.clip(i - q * NBH_, 0, NBH_ - 1), 0)

  out = pl.pallas_call(
      _b_body,
      grid=(NB_,),
      in_specs=[
          pl.BlockSpec((BT_, C_), qmap(0)),
          pl.BlockSpec((BT_, C_), qmap(1)),
          pl.BlockSpec((BT_, C_), qmap(2)),
          pl.BlockSpec((BT_, C_), qmap(3)),
          pl.BlockSpec((C_, C_), lambda i: (0, 0)),
          pl.BlockSpec((8, C_), lambda i: (0, 0)),
          pl.BlockSpec((8, C_), lambda i: (0, 0)),
          pl.BlockSpec((8, C_), lambda i: (0, 0)),
          pl.BlockSpec((8, C_), lambda i: (0, 0)),
          pl.BlockSpec((8, C_), lambda i: (0, 0)),
          pl.BlockSpec((8, C_), lambda i: (0, 0)),
      ],
      out_specs=pl.BlockSpec((BT_, C_), lambda i: (i, 0)),
      out_shape=jax.ShapeDtypeStruct((F_, C_), jnp.float32),
  )(*vhs, depth_weights, bb8, *sts, gb8)

  return out


# speedup vs baseline: 1.1595x; 1.1595x over previous
import functools

import jax
import jax.numpy as jnp
from jax import lax
from jax.experimental import pallas as pl
from jax.experimental.pallas import tpu as pltpu
from jax.experimental.pallas import tpu_sc as plsc

N_ = 10000
F_ = 320000
C_ = 128
NC_ = 2
NS_ = 16
NW_ = NC_ * NS_
CHUNK_ = 128
FH_ = F_ // 2
NFULL_ = 39
NEXTRA_ = (FH_ - NW_ * NFULL_ * CHUNK_) // CHUNK_
ROWS_W_ = NFULL_ * CHUNK_

BT_ = 4000
NB_ = F_ // BT_
NBH_ = FH_ // BT_


def _c_body(inp, sw, t):
  x = inp[...]
  t[pl.ds(0, N_), :] = x * sw[0, :][None, :]
  t[pl.ds(N_, N_), :] = x * sw[1, :][None, :]
  t[pl.ds(2 * N_, N_), :] = x * sw[2, :][None, :]


def _sc_body(base0, t_hbm, i0_hbm, i1_hbm, i2_hbm, v2f_hbm,
             iv0, iv1, iv2,
             ra0, ra1, ra2, rb0, rb1, rb2, sa, sb):
  wid = lax.axis_index("s") * NC_ + lax.axis_index("c")
  wbase = wid * ROWS_W_
  gbase = base0 + wbase

  pltpu.sync_copy(i0_hbm.at[pl.ds(gbase, ROWS_W_)], iv0)
  pltpu.sync_copy(i1_hbm.at[pl.ds(gbase, ROWS_W_)], iv1)
  pltpu.sync_copy(i2_hbm.at[pl.ds(gbase, ROWS_W_)], iv2)

  def issue(bufs, sem, j):
    off = j * CHUNK_
    pltpu.async_copy(t_hbm.at[iv0.at[pl.ds(off, CHUNK_)]], bufs[0], sem)
    pltpu.async_copy(t_hbm.at[iv1.at[pl.ds(off, CHUNK_)]], bufs[1], sem)
    pltpu.async_copy(t_hbm.at[iv2.at[pl.ds(off, CHUNK_)]], bufs[2], sem)

  def drain(bufs, sem, j):
    off = j * CHUNK_
    pltpu.make_async_copy(t_hbm.at[iv0.at[pl.ds(off, CHUNK_)]], bufs[0],
                          sem).wait()
    pltpu.make_async_copy(t_hbm.at[iv1.at[pl.ds(off, CHUNK_)]], bufs[1],
                          sem).wait()
    pltpu.make_async_copy(t_hbm.at[iv2.at[pl.ds(off, CHUNK_)]], bufs[2],
                          sem).wait()

  def combine(bufs):
    def row(r, carry):
      for s in range(C_ // 16):
        sl = pl.ds(s * 16, 16)
        bufs[0][r, sl] = bufs[0][r, sl] + bufs[1][r, sl] + bufs[2][r, sl]
      return carry

    lax.fori_loop(0, CHUNK_, row, 0)

  def store(bufs, base):
    pltpu.sync_copy(bufs[0], v2f_hbm.at[pl.ds(base, CHUNK_)])

  bufs_a = (ra0, ra1, ra2)
  bufs_b = (rb0, rb1, rb2)

  issue(bufs_a, sa, 0)

  def body(i, carry):
    j0 = 2 * i

    @pl.when(j0 + 1 < NFULL_)
    def _():
      issue(bufs_b, sb, j0 + 1)

    drain(bufs_a, sa, j0)
    combine(bufs_a)
    store(bufs_a, wbase + j0 * CHUNK_)

    @pl.when(j0 + 2 < NFULL_)
    def _():
      issue(bufs_a, sa, j0 + 2)

    @pl.when(j0 + 1 < NFULL_)
    def _():
      drain(bufs_b, sb, j0 + 1)
      combine(bufs_b)
      store(bufs_b, wbase + (j0 + 1) * CHUNK_)
    return carry

  lax.fori_loop(0, (NFULL_ + 1) // 2, body, 0)

  @pl.when(wid < NEXTRA_)
  def _():
    base = (NW_ * NFULL_ + wid) * CHUNK_
    pltpu.sync_copy(i0_hbm.at[pl.ds(base0 + base, CHUNK_)],
                    iv0.at[pl.ds(0, CHUNK_)])
    pltpu.sync_copy(i1_hbm.at[pl.ds(base0 + base, CHUNK_)],
                    iv1.at[pl.ds(0, CHUNK_)])
    pltpu.sync_copy(i2_hbm.at[pl.ds(base0 + base, CHUNK_)],
                    iv2.at[pl.ds(0, CHUNK_)])
    issue(bufs_a, sa, 0)
    drain(bufs_a, sa, 0)
    combine(bufs_a)
    store(bufs_a, base)


@functools.lru_cache(maxsize=None)
def _get_sc_combine(base0):
  return pl.kernel(
    out_type=jax.ShapeDtypeStruct((FH_, C_), jnp.float32),
    mesh=plsc.VectorSubcoreMesh(core_axis_name="c", subcore_axis_name="s"),
    scratch_types=[
        pltpu.VMEM((ROWS_W_,), jnp.int32),
        pltpu.VMEM((ROWS_W_,), jnp.int32),
        pltpu.VMEM((ROWS_W_,), jnp.int32),
        pltpu.VMEM((CHUNK_, C_), jnp.float32),
        pltpu.VMEM((CHUNK_, C_), jnp.float32),
        pltpu.VMEM((CHUNK_, C_), jnp.float32),
        pltpu.VMEM((CHUNK_, C_), jnp.float32),
        pltpu.VMEM((CHUNK_, C_), jnp.float32),
        pltpu.VMEM((CHUNK_, C_), jnp.float32),
        pltpu.SemaphoreType.DMA,
        pltpu.SemaphoreType.DMA,
    ],
  )(functools.partial(_sc_body, base0))


def _a_body(v2f, dw, bb, stats):
  acc = jnp.dot(v2f[...], dw[...], preferred_element_type=jnp.float32)
  acc = acc + bb[0, :][None, :]
  r = jnp.maximum(acc, 0.0)
  s = jnp.sum(r, axis=0)
  s2 = jnp.sum(r * r, axis=0)
  upd = jnp.concatenate(
      [s[None, :], s2[None, :], jnp.zeros((6, C_), jnp.float32)], axis=0)

  @pl.when(pl.program_id(0) == 0)
  def _():
    stats[...] = upd

  @pl.when(pl.program_id(0) != 0)
  def _():
    stats[...] = stats[...] + upd


def _b_body(vh1, vh2, dw, bb, st1, st2, gb, out):
  i = pl.program_id(0)
  x = jnp.where(i < NBH_, vh1[...], vh2[...])
  acc = jnp.dot(x, dw[...], preferred_element_type=jnp.float32)
  acc = acc + bb[0, :][None, :]
  r = jnp.maximum(acc, 0.0)
  s = st1[0, :] + st2[0, :]
  s2 = st1[1, :] + st2[1, :]
  mean = s / F_
  var = s2 / F_ - mean * mean
  inv = gb[0, :] / jnp.sqrt(var + 1e-5)
  out[...] = (r - mean[None, :]) * inv[None, :] + gb[1, :][None, :]


def kernel(inputs, face, spatial_weights, depth_weights, biases,
           bn_gamma, bn_beta):
  face32 = face.astype(jnp.int32)
  ft = face32.T
  i0 = ft[0]
  i1 = ft[1] + N_
  i2 = ft[2] + 2 * N_

  sw8 = jnp.pad(spatial_weights[:, :, 0], ((0, 5), (0, 0)))
  bb8 = jnp.pad(biases, ((0, 7), (0, 0)))
  gb8 = jnp.pad(jnp.stack([bn_gamma, bn_beta]), ((0, 6), (0, 0)))

  t = pl.pallas_call(
      _c_body,
      in_specs=[
          pl.BlockSpec((N_, C_), lambda: (0, 0)),
          pl.BlockSpec((8, C_), lambda: (0, 0)),
      ],
      out_specs=pl.BlockSpec((3 * N_, C_), lambda: (0, 0)),
      out_shape=jax.ShapeDtypeStruct((3 * N_, C_), jnp.float32),
  )(inputs, sw8)

  vh1 = _get_sc_combine(0)(t, i0, i1, i2)
  vh2 = _get_sc_combine(FH_)(t, i0, i1, i2)

  def run_a(vh):
    return pl.pallas_call(
        _a_body,
        grid=(NBH_,),
        in_specs=[
            pl.BlockSpec((BT_, C_), lambda i: (i, 0)),
            pl.BlockSpec((C_, C_), lambda i: (0, 0)),
            pl.BlockSpec((8, C_), lambda i: (0, 0)),
        ],
        out_specs=pl.BlockSpec((8, C_), lambda i: (0, 0)),
        out_shape=jax.ShapeDtypeStruct((8, C_), jnp.float32),
    )(vh, depth_weights, bb8)

  st1 = run_a(vh1)
  st2 = run_a(vh2)

  out = pl.pallas_call(
      _b_body,
      grid=(NB_,),
      in_specs=[
          pl.BlockSpec((BT_, C_),
                       lambda i: (jnp.minimum(i, NBH_ - 1), 0)),
          pl.BlockSpec((BT_, C_),
                       lambda i: (jnp.maximum(i - NBH_, 0), 0)),
          pl.BlockSpec((C_, C_), lambda i: (0, 0)),
          pl.BlockSpec((8, C_), lambda i: (0, 0)),
          pl.BlockSpec((8, C_), lambda i: (0, 0)),
          pl.BlockSpec((8, C_), lambda i: (0, 0)),
          pl.BlockSpec((8, C_), lambda i: (0, 0)),
      ],
      out_specs=pl.BlockSpec((BT_, C_), lambda i: (i, 0)),
      out_shape=jax.ShapeDtypeStruct((F_, C_), jnp.float32),
  )(vh1, vh2, depth_weights, bb8, st1, st2, gb8)

  return out


# speedup vs baseline: 1.2437x; 1.0726x over previous
import functools

import jax
import jax.numpy as jnp
from jax import lax
from jax.experimental import pallas as pl
from jax.experimental.pallas import tpu as pltpu
from jax.experimental.pallas import tpu_sc as plsc

N_ = 10000
F_ = 320000
C_ = 128
NC_ = 2
NS_ = 16
NW_ = NC_ * NS_
CHUNK_ = 128
FH_ = F_ // 2
NFULL_ = 39
NEXTRA_ = (FH_ - NW_ * NFULL_ * CHUNK_) // CHUNK_
ROWS_W_ = NFULL_ * CHUNK_

BT_ = 8000
NB_ = F_ // BT_
NBH_ = FH_ // BT_


def _c_body(inp, sw, t):
  x = inp[...]
  t[pl.ds(0, N_), :] = x * sw[0, :][None, :]
  t[pl.ds(N_, N_), :] = x * sw[1, :][None, :]
  t[pl.ds(2 * N_, N_), :] = x * sw[2, :][None, :]


def _sc_body(base0, t_hbm, i0_hbm, i1_hbm, i2_hbm, v2f_hbm,
             iv0, iv1, iv2,
             ra0, ra1, ra2, rb0, rb1, rb2, sa, sb):
  wid = lax.axis_index("s") * NC_ + lax.axis_index("c")
  wbase = wid * ROWS_W_
  gbase = base0 + wbase

  pltpu.sync_copy(i0_hbm.at[pl.ds(gbase, ROWS_W_)], iv0)
  pltpu.sync_copy(i1_hbm.at[pl.ds(gbase, ROWS_W_)], iv1)
  pltpu.sync_copy(i2_hbm.at[pl.ds(gbase, ROWS_W_)], iv2)

  def issue(bufs, sem, j):
    off = j * CHUNK_
    pltpu.async_copy(t_hbm.at[iv0.at[pl.ds(off, CHUNK_)]], bufs[0], sem)
    pltpu.async_copy(t_hbm.at[iv1.at[pl.ds(off, CHUNK_)]], bufs[1], sem)
    pltpu.async_copy(t_hbm.at[iv2.at[pl.ds(off, CHUNK_)]], bufs[2], sem)

  def drain(bufs, sem, j):
    off = j * CHUNK_
    pltpu.make_async_copy(t_hbm.at[iv0.at[pl.ds(off, CHUNK_)]], bufs[0],
                          sem).wait()
    pltpu.make_async_copy(t_hbm.at[iv1.at[pl.ds(off, CHUNK_)]], bufs[1],
                          sem).wait()
    pltpu.make_async_copy(t_hbm.at[iv2.at[pl.ds(off, CHUNK_)]], bufs[2],
                          sem).wait()

  def combine(bufs):
    def row(r, carry):
      for s in range(C_ // 16):
        sl = pl.ds(s * 16, 16)
        bufs[0][r, sl] = bufs[0][r, sl] + bufs[1][r, sl] + bufs[2][r, sl]
      return carry

    lax.fori_loop(0, CHUNK_, row, 0)

  def store(bufs, base):
    pltpu.sync_copy(bufs[0], v2f_hbm.at[pl.ds(base, CHUNK_)])

  bufs_a = (ra0, ra1, ra2)
  bufs_b = (rb0, rb1, rb2)

  issue(bufs_a, sa, 0)

  def body(i, carry):
    j0 = 2 * i

    @pl.when(j0 + 1 < NFULL_)
    def _():
      issue(bufs_b, sb, j0 + 1)

    drain(bufs_a, sa, j0)
    combine(bufs_a)
    store(bufs_a, wbase + j0 * CHUNK_)

    @pl.when(j0 + 2 < NFULL_)
    def _():
      issue(bufs_a, sa, j0 + 2)

    @pl.when(j0 + 1 < NFULL_)
    def _():
      drain(bufs_b, sb, j0 + 1)
      combine(bufs_b)
      store(bufs_b, wbase + (j0 + 1) * CHUNK_)
    return carry

  lax.fori_loop(0, (NFULL_ + 1) // 2, body, 0)

  @pl.when(wid < NEXTRA_)
  def _():
    base = (NW_ * NFULL_ + wid) * CHUNK_
    pltpu.sync_copy(i0_hbm.at[pl.ds(base0 + base, CHUNK_)],
                    iv0.at[pl.ds(0, CHUNK_)])
    pltpu.sync_copy(i1_hbm.at[pl.ds(base0 + base, CHUNK_)],
                    iv1.at[pl.ds(0, CHUNK_)])
    pltpu.sync_copy(i2_hbm.at[pl.ds(base0 + base, CHUNK_)],
                    iv2.at[pl.ds(0, CHUNK_)])
    issue(bufs_a, sa, 0)
    drain(bufs_a, sa, 0)
    combine(bufs_a)
    store(bufs_a, base)


@functools.lru_cache(maxsize=None)
def _get_sc_combine(base0):
  return pl.kernel(
    out_type=jax.ShapeDtypeStruct((FH_, C_), jnp.float32),
    mesh=plsc.VectorSubcoreMesh(core_axis_name="c", subcore_axis_name="s"),
    scratch_types=[
        pltpu.VMEM((ROWS_W_,), jnp.int32),
        pltpu.VMEM((ROWS_W_,), jnp.int32),
        pltpu.VMEM((ROWS_W_,), jnp.int32),
        pltpu.VMEM((CHUNK_, C_), jnp.float32),
        pltpu.VMEM((CHUNK_, C_), jnp.float32),
        pltpu.VMEM((CHUNK_, C_), jnp.float32),
        pltpu.VMEM((CHUNK_, C_), jnp.float32),
        pltpu.VMEM((CHUNK_, C_), jnp.float32),
        pltpu.VMEM((CHUNK_, C_), jnp.float32),
        pltpu.SemaphoreType.DMA,
        pltpu.SemaphoreType.DMA,
    ],
  )(functools.partial(_sc_body, base0))


def _a_body(v2f, dw, bb, stats):
  acc = jnp.dot(v2f[...], dw[...], preferred_element_type=jnp.float32)
  acc = acc + bb[0, :][None, :]
  r = jnp.maximum(acc, 0.0)
  s = jnp.sum(r, axis=0)
  s2 = jnp.sum(r * r, axis=0)
  upd = jnp.concatenate(
      [s[None, :], s2[None, :], jnp.zeros((6, C_), jnp.float32)], axis=0)

  @pl.when(pl.program_id(0) == 0)
  def _():
    stats[...] = upd

  @pl.when(pl.program_id(0) != 0)
  def _():
    stats[...] = stats[...] + upd


def _b_body(vh1, vh2, dw, bb, st1, st2, gb, out):
  i = pl.program_id(0)
  x = jnp.where(i < NBH_, vh1[...], vh2[...])
  acc = jnp.dot(x, dw[...], preferred_element_type=jnp.float32)
  acc = acc + bb[0, :][None, :]
  r = jnp.maximum(acc, 0.0)
  s = st1[0, :] + st2[0, :]
  s2 = st1[1, :] + st2[1, :]
  mean = s / F_
  var = s2 / F_ - mean * mean
  inv = gb[0, :] / jnp.sqrt(var + 1e-5)
  out[...] = (r - mean[None, :]) * inv[None, :] + gb[1, :][None, :]


def kernel(inputs, face, spatial_weights, depth_weights, biases,
           bn_gamma, bn_beta):
  face32 = face.astype(jnp.int32)
  ft = face32.T
  i0 = ft[0]
  i1 = ft[1] + N_
  i2 = ft[2] + 2 * N_

  sw8 = jnp.pad(spatial_weights[:, :, 0], ((0, 5), (0, 0)))
  bb8 = jnp.pad(biases, ((0, 7), (0, 0)))
  gb8 = jnp.pad(jnp.stack([bn_gamma, bn_beta]), ((0, 6), (0, 0)))

  t = pl.pallas_call(
      _c_body,
      in_specs=[
          pl.BlockSpec((N_, C_), lambda: (0, 0)),
          pl.BlockSpec((8, C_), lambda: (0, 0)),
      ],
      out_specs=pl.BlockSpec((3 * N_, C_), lambda: (0, 0)),
      out_shape=jax.ShapeDtypeStruct((3 * N_, C_), jnp.float32),
  )(inputs, sw8)

  vh1 = _get_sc_combine(0)(t, i0, i1, i2)
  vh2 = _get_sc_combine(FH_)(t, i0, i1, i2)

  def run_a(vh):
    return pl.pallas_call(
        _a_body,
        grid=(NBH_,),
        in_specs=[
            pl.BlockSpec((BT_, C_), lambda i: (i, 0)),
            pl.BlockSpec((C_, C_), lambda i: (0, 0)),
            pl.BlockSpec((8, C_), lambda i: (0, 0)),
        ],
        out_specs=pl.BlockSpec((8, C_), lambda i: (0, 0)),
        out_shape=jax.ShapeDtypeStruct((8, C_), jnp.float32),
    )(vh, depth_weights, bb8)

  st1 = run_a(vh1)
  st2 = run_a(vh2)

  out = pl.pallas_call(
      _b_body,
      grid=(NB_,),
      in_specs=[
          pl.BlockSpec((BT_, C_),
                       lambda i: (jnp.minimum(i, NBH_ - 1), 0)),
          pl.BlockSpec((BT_, C_),
                       lambda i: (jnp.maximum(i - NBH_, 0), 0)),
          pl.BlockSpec((C_, C_), lambda i: (0, 0)),
          pl.BlockSpec((8, C_), lambda i: (0, 0)),
          pl.BlockSpec((8, C_), lambda i: (0, 0)),
          pl.BlockSpec((8, C_), lambda i: (0, 0)),
          pl.BlockSpec((8, C_), lambda i: (0, 0)),
      ],
      out_specs=pl.BlockSpec((BT_, C_), lambda i: (i, 0)),
      out_shape=jax.ShapeDtypeStruct((F_, C_), jnp.float32),
  )(vh1, vh2, depth_weights, bb8, st1, st2, gb8)

  return out


# speedup vs baseline: 1.2739x; 1.0243x over previous
import functools

import jax
import jax.numpy as jnp
from jax import lax
from jax.experimental import pallas as pl
from jax.experimental.pallas import tpu as pltpu
from jax.experimental.pallas import tpu_sc as plsc

N_ = 10000
F_ = 320000
C_ = 128
NC_ = 2
NS_ = 16
NW_ = NC_ * NS_
CHUNK_ = 128
FH_ = F_ // 2
NFULL_ = 39
NEXTRA_ = (FH_ - NW_ * NFULL_ * CHUNK_) // CHUNK_
ROWS_W_ = NFULL_ * CHUNK_

BT_ = 16000
NB_ = F_ // BT_
NBH_ = FH_ // BT_


def _c_body(inp, sw, t):
  x = inp[...]
  t[pl.ds(0, N_), :] = x * sw[0, :][None, :]
  t[pl.ds(N_, N_), :] = x * sw[1, :][None, :]
  t[pl.ds(2 * N_, N_), :] = x * sw[2, :][None, :]


def _sc_body(base0, t_hbm, i0_hbm, i1_hbm, i2_hbm, v2f_hbm,
             iv0, iv1, iv2,
             ra0, ra1, ra2, rb0, rb1, rb2, sa, sb):
  wid = lax.axis_index("s") * NC_ + lax.axis_index("c")
  wbase = wid * ROWS_W_
  gbase = base0 + wbase

  pltpu.sync_copy(i0_hbm.at[pl.ds(gbase, ROWS_W_)], iv0)
  pltpu.sync_copy(i1_hbm.at[pl.ds(gbase, ROWS_W_)], iv1)
  pltpu.sync_copy(i2_hbm.at[pl.ds(gbase, ROWS_W_)], iv2)

  def issue(bufs, sem, j):
    off = j * CHUNK_
    pltpu.async_copy(t_hbm.at[iv0.at[pl.ds(off, CHUNK_)]], bufs[0], sem)
    pltpu.async_copy(t_hbm.at[iv1.at[pl.ds(off, CHUNK_)]], bufs[1], sem)
    pltpu.async_copy(t_hbm.at[iv2.at[pl.ds(off, CHUNK_)]], bufs[2], sem)

  def drain(bufs, sem, j):
    off = j * CHUNK_
    pltpu.make_async_copy(t_hbm.at[iv0.at[pl.ds(off, CHUNK_)]], bufs[0],
                          sem).wait()
    pltpu.make_async_copy(t_hbm.at[iv1.at[pl.ds(off, CHUNK_)]], bufs[1],
                          sem).wait()
    pltpu.make_async_copy(t_hbm.at[iv2.at[pl.ds(off, CHUNK_)]], bufs[2],
                          sem).wait()

  def combine(bufs):
    def row(r, carry):
      for s in range(C_ // 16):
        sl = pl.ds(s * 16, 16)
        bufs[0][r, sl] = bufs[0][r, sl] + bufs[1][r, sl] + bufs[2][r, sl]
      return carry

    lax.fori_loop(0, CHUNK_, row, 0)

  def store(bufs, base):
    pltpu.sync_copy(bufs[0], v2f_hbm.at[pl.ds(base, CHUNK_)])

  bufs_a = (ra0, ra1, ra2)
  bufs_b = (rb0, rb1, rb2)

  issue(bufs_a, sa, 0)

  def body(i, carry):
    j0 = 2 * i

    @pl.when(j0 + 1 < NFULL_)
    def _():
      issue(bufs_b, sb, j0 + 1)

    drain(bufs_a, sa, j0)
    combine(bufs_a)
    store(bufs_a, wbase + j0 * CHUNK_)

    @pl.when(j0 + 2 < NFULL_)
    def _():
      issue(bufs_a, sa, j0 + 2)

    @pl.when(j0 + 1 < NFULL_)
    def _():
      drain(bufs_b, sb, j0 + 1)
      combine(bufs_b)
      store(bufs_b, wbase + (j0 + 1) * CHUNK_)
    return carry

  lax.fori_loop(0, (NFULL_ + 1) // 2, body, 0)

  @pl.when(wid < NEXTRA_)
  def _():
    base = (NW_ * NFULL_ + wid) * CHUNK_
    pltpu.sync_copy(i0_hbm.at[pl.ds(base0 + base, CHUNK_)],
                    iv0.at[pl.ds(0, CHUNK_)])
    pltpu.sync_copy(i1_hbm.at[pl.ds(base0 + base, CHUNK_)],
                    iv1.at[pl.ds(0, CHUNK_)])
    pltpu.sync_copy(i2_hbm.at[pl.ds(base0 + base, CHUNK_)],
                    iv2.at[pl.ds(0, CHUNK_)])
    issue(bufs_a, sa, 0)
    drain(bufs_a, sa, 0)
    combine(bufs_a)
    store(bufs_a, base)


@functools.lru_cache(maxsize=None)
def _get_sc_combine(base0):
  return pl.kernel(
    out_type=jax.ShapeDtypeStruct((FH_, C_), jnp.float32),
    mesh=plsc.VectorSubcoreMesh(core_axis_name="c", subcore_axis_name="s"),
    scratch_types=[
        pltpu.VMEM((ROWS_W_,), jnp.int32),
        pltpu.VMEM((ROWS_W_,), jnp.int32),
        pltpu.VMEM((ROWS_W_,), jnp.int32),
        pltpu.VMEM((CHUNK_, C_), jnp.float32),
        pltpu.VMEM((CHUNK_, C_), jnp.float32),
        pltpu.VMEM((CHUNK_, C_), jnp.float32),
        pltpu.VMEM((CHUNK_, C_), jnp.float32),
        pltpu.VMEM((CHUNK_, C_), jnp.float32),
        pltpu.VMEM((CHUNK_, C_), jnp.float32),
        pltpu.SemaphoreType.DMA,
        pltpu.SemaphoreType.DMA,
    ],
  )(functools.partial(_sc_body, base0))


def _a_body(v2f, dw, bb, stats):
  acc = jnp.dot(v2f[...], dw[...], preferred_element_type=jnp.float32)
  acc = acc + bb[0, :][None, :]
  r = jnp.maximum(acc, 0.0)
  s = jnp.sum(r, axis=0)
  s2 = jnp.sum(r * r, axis=0)
  upd = jnp.concatenate(
      [s[None, :], s2[None, :], jnp.zeros((6, C_), jnp.float32)], axis=0)

  @pl.when(pl.program_id(0) == 0)
  def _():
    stats[...] = upd

  @pl.when(pl.program_id(0) != 0)
  def _():
    stats[...] = stats[...] + upd


def _b_body(vh1, vh2, dw, bb, st1, st2, gb, out):
  i = pl.program_id(0)
  x = jnp.where(i < NBH_, vh1[...], vh2[...])
  acc = jnp.dot(x, dw[...], preferred_element_type=jnp.float32)
  acc = acc + bb[0, :][None, :]
  r = jnp.maximum(acc, 0.0)
  s = st1[0, :] + st2[0, :]
  s2 = st1[1, :] + st2[1, :]
  mean = s / F_
  var = s2 / F_ - mean * mean
  inv = gb[0, :] / jnp.sqrt(var + 1e-5)
  out[...] = (r - mean[None, :]) * inv[None, :] + gb[1, :][None, :]


def kernel(inputs, face, spatial_weights, depth_weights, biases,
           bn_gamma, bn_beta):
  face32 = face.astype(jnp.int32)
  ft = face32.T
  i0 = ft[0]
  i1 = ft[1] + N_
  i2 = ft[2] + 2 * N_

  sw8 = jnp.pad(spatial_weights[:, :, 0], ((0, 5), (0, 0)))
  bb8 = jnp.pad(biases, ((0, 7), (0, 0)))
  gb8 = jnp.pad(jnp.stack([bn_gamma, bn_beta]), ((0, 6), (0, 0)))

  t = pl.pallas_call(
      _c_body,
      in_specs=[
          pl.BlockSpec((N_, C_), lambda: (0, 0)),
          pl.BlockSpec((8, C_), lambda: (0, 0)),
      ],
      out_specs=pl.BlockSpec((3 * N_, C_), lambda: (0, 0)),
      out_shape=jax.ShapeDtypeStruct((3 * N_, C_), jnp.float32),
  )(inputs, sw8)

  vh1 = _get_sc_combine(0)(t, i0, i1, i2)
  vh2 = _get_sc_combine(FH_)(t, i0, i1, i2)

  def run_a(vh):
    return pl.pallas_call(
        _a_body,
        grid=(NBH_,),
        in_specs=[
            pl.BlockSpec((BT_, C_), lambda i: (i, 0)),
            pl.BlockSpec((C_, C_), lambda i: (0, 0)),
            pl.BlockSpec((8, C_), lambda i: (0, 0)),
        ],
        out_specs=pl.BlockSpec((8, C_), lambda i: (0, 0)),
        out_shape=jax.ShapeDtypeStruct((8, C_), jnp.float32),
    )(vh, depth_weights, bb8)

  st1 = run_a(vh1)
  st2 = run_a(vh2)

  out = pl.pallas_call(
      _b_body,
      grid=(NB_,),
      in_specs=[
          pl.BlockSpec((BT_, C_),
                       lambda i: (jnp.minimum(i, NBH_ - 1), 0)),
          pl.BlockSpec((BT_, C_),
                       lambda i: (jnp.maximum(i - NBH_, 0), 0)),
          pl.BlockSpec((C_, C_), lambda i: (0, 0)),
          pl.BlockSpec((8, C_), lambda i: (0, 0)),
          pl.BlockSpec((8, C_), lambda i: (0, 0)),
          pl.BlockSpec((8, C_), lambda i: (0, 0)),
          pl.BlockSpec((8, C_), lambda i: (0, 0)),
      ],
      out_specs=pl.BlockSpec((BT_, C_), lambda i: (i, 0)),
      out_shape=jax.ShapeDtypeStruct((F_, C_), jnp.float32),
  )(vh1, vh2, depth_weights, bb8, st1, st2, gb8)

  return out
